# baseline (speedup 1.0000x reference)
import jax
import jax.numpy as jnp
from jax import lax
from jax.experimental import pallas as pl
from jax.experimental.pallas import tpu as pltpu
from jax.experimental.pallas import tpu_sc as plsc

_NU = 1000000
_NI = 100000
_D = 32
_B = 16384

_INFO = plsc.get_sparse_core_info()
_NC = _INFO.num_cores
_NS = _INFO.num_subcores
_NW = _NC * _NS
_BPW = _B // _NW
_L = 16
_NCH = _BPW // _L
_CIT = 128


def _body(z_hbm, tu_hbm, ti_hbm, uy_hbm, zig_hbm, iyg_hbm, out_hbm,
          tu_v, ti_v, it_idx, stage_zi, stage_iy,
          zu_v, uy_v, zi_c, iy_c, out_v, sem_it, sem_u):
    wid = lax.axis_index("s") * _NC + lax.axis_index("c")
    base = wid * _BPW

    pltpu.sync_copy(tu_hbm.at[pl.ds(base, _BPW)], tu_v)
    pltpu.sync_copy(ti_hbm.at[pl.ds(base, _BPW)], ti_v)

    for c in range(_BPW // _CIT):
        for g in range(_CIT // _L):
            sl = pl.ds(c * _CIT + g * _L, _L)
            it_idx[pl.ds(g * _L, _L)] = lax.shift_right_logical(ti_v[sl], 2)
        cp_a = pltpu.async_copy(zig_hbm.at[it_idx], stage_zi, sem_it)
        cp_b = pltpu.async_copy(iyg_hbm.at[it_idx], stage_iy, sem_it)
        cp_a.wait()
        cp_b.wait()

        def extract(g, _, c=c):
            ti16 = ti_v[pl.ds(c * _CIT + g * _L, _L)]
            q16 = (ti16 & 3) << 5
            for k in range(_L):
                q = q16[k]
                row = g * _L + k
                dr = c * (_CIT // 4) + g * 4 + k // 4
                dc = (k & 3) * _D
                zi_c[dr, pl.ds(dc, _L)] = stage_zi[row, pl.ds(q, _L)]
                zi_c[dr, pl.ds(dc + _L, _L)] = stage_zi[row, pl.ds(q + _L, _L)]
                iy_c[dr, pl.ds(dc, _L)] = stage_iy[row, pl.ds(q, _L)]
                iy_c[dr, pl.ds(dc + _L, _L)] = stage_iy[row, pl.ds(q + _L, _L)]
            return 0

        lax.fori_loop(0, _CIT // _L, extract, 0)

    def fire(ch, _):
        tu16 = tu_v[pl.ds(ch * _L, _L)]
        for k in range(_L):
            r = tu16[k]
            jr = ch * 4 + k // 4
            jc = (k & 3) * _D
            pltpu.async_copy(z_hbm.at[r], zu_v.at[jr, pl.ds(jc, _D)], sem_u)
            pltpu.async_copy(uy_hbm.at[r], uy_v.at[jr, pl.ds(jc, _D)], sem_u)
        return 0

    lax.fori_loop(0, _NCH, fire, 0)

    def drain(ch, _):
        for k in range(_L):
            jr = ch * 4 + k // 4
            jc = (k & 3) * _D
            pltpu.make_async_copy(
                z_hbm.at[0], zu_v.at[jr, pl.ds(jc, _D)], sem_u).wait()
            pltpu.make_async_copy(
                uy_hbm.at[0], uy_v.at[jr, pl.ds(jc, _D)], sem_u).wait()
        return 0

    lax.fori_loop(0, _NCH, drain, 0)

    lane = lax.broadcasted_iota(jnp.int32, (_L,), 0)

    def dot(ch, _):
        acc = jnp.zeros((_L,), jnp.float32)
        for k in range(_L):
            jr = ch * 4 + k // 4
            jc = (k & 3) * _D
            u = (zu_v[jr, pl.ds(jc, _L)] * zi_c[jr, pl.ds(jc, _L)]
                 + zu_v[jr, pl.ds(jc + _L, _L)] * zi_c[jr, pl.ds(jc + _L, _L)]
                 + uy_v[jr, pl.ds(jc, _L)] * iy_c[jr, pl.ds(jc, _L)]
                 + uy_v[jr, pl.ds(jc + _L, _L)] * iy_c[jr, pl.ds(jc + _L, _L)])
            acc = jnp.where(lane == k, jnp.sum(u), acc)
        out_v[pl.ds(ch * _L, _L)] = acc
        return 0

    lax.fori_loop(0, _NCH, dot, 0)

    pltpu.sync_copy(out_v, out_hbm.at[pl.ds(base, _BPW)])


@jax.jit
def _mf(z, tu, ti, uy, zig, iyg):
    mesh = plsc.VectorSubcoreMesh(core_axis_name="c", subcore_axis_name="s")
    return pl.kernel(
        _body,
        mesh=mesh,
        compiler_params=pltpu.CompilerParams(
            needs_layout_passes=False, use_tc_tiling_on_sc=True),
        out_type=jax.ShapeDtypeStruct((_B,), jnp.float32),
        scratch_types=[
            pltpu.VMEM((_BPW,), jnp.int32),
            pltpu.VMEM((_BPW,), jnp.int32),
            pltpu.VMEM((_CIT,), jnp.int32),
            pltpu.VMEM((_CIT, 128), jnp.float32),
            pltpu.VMEM((_CIT, 128), jnp.float32),
            pltpu.VMEM((_BPW // 4, 128), jnp.float32),
            pltpu.VMEM((_BPW // 4, 128), jnp.float32),
            pltpu.VMEM((_BPW // 4, 128), jnp.float32),
            pltpu.VMEM((_BPW // 4, 128), jnp.float32),
            pltpu.VMEM((_BPW,), jnp.float32),
            pltpu.SemaphoreType.DMA,
            pltpu.SemaphoreType.DMA,
        ],
    )(z, tu, ti, uy, zig, iyg)


def kernel(Z, Tu, Ti, uY, iY):
    zig = Z[_NU:].reshape(_NI // 4, 128)
    iyg = iY.reshape(_NI // 4, 128)
    return _mf(Z, Tu.astype(jnp.int32), Ti.astype(jnp.int32), uY, zig, iyg)


# speedup vs baseline: 1.1181x; 1.1181x over previous
import jax
import jax.numpy as jnp
from jax import lax
from jax.experimental import pallas as pl
from jax.experimental.pallas import tpu as pltpu
from jax.experimental.pallas import tpu_sc as plsc

_NU = 1000000
_NI = 100000
_D = 32
_B = 16384

_INFO = plsc.get_sparse_core_info()
_NC = _INFO.num_cores
_NS = _INFO.num_subcores
_NW = _NC * _NS
_BPW = _B // _NW
_L = 16
_NCH = _BPW // _L


def _body(z_hbm, tu_hbm, ti_hbm, uy_hbm, iy_hbm, out_hbm,
          tu_v, ti_v, zu_v, uy_v, zi_v, iy_v, out_v, sem):
    wid = lax.axis_index("s") * _NC + lax.axis_index("c")
    base = wid * _BPW

    pltpu.sync_copy(tu_hbm.at[pl.ds(base, _BPW)], tu_v)
    pltpu.sync_copy(ti_hbm.at[pl.ds(base, _BPW)], ti_v)

    def fire(ch, _):
        tu16 = tu_v[pl.ds(ch * _L, _L)]
        ti16 = ti_v[pl.ds(ch * _L, _L)]
        tz16 = ti16 + _NU
        for k in range(_L):
            r = tu16[k]
            i = ti16[k]
            iz = tz16[k]
            jr = ch * 4 + k // 4
            jc = (k & 3) * _D
            dst = (jr, pl.ds(jc, _D))
            pltpu.async_copy(z_hbm.at[r], zu_v.at[dst], sem)
            pltpu.async_copy(uy_hbm.at[r], uy_v.at[dst], sem)
            pltpu.async_copy(z_hbm.at[iz], zi_v.at[dst], sem)
            pltpu.async_copy(iy_hbm.at[i], iy_v.at[dst], sem)
        return 0

    lax.fori_loop(0, _NCH, fire, 0)

    def drain(ch, _):
        for k in range(_L):
            jr = ch * 4 + k // 4
            jc = (k & 3) * _D
            dst = (jr, pl.ds(jc, _D))
            pltpu.make_async_copy(z_hbm.at[0], zu_v.at[dst], sem).wait()
            pltpu.make_async_copy(z_hbm.at[0], uy_v.at[dst], sem).wait()
            pltpu.make_async_copy(z_hbm.at[0], zi_v.at[dst], sem).wait()
            pltpu.make_async_copy(z_hbm.at[0], iy_v.at[dst], sem).wait()
        return 0

    lax.fori_loop(0, _NCH, drain, 0)

    lane = lax.broadcasted_iota(jnp.int32, (_L,), 0)

    def dot(ch, _):
        acc = jnp.zeros((_L,), jnp.float32)
        for k in range(_L):
            jr = ch * 4 + k // 4
            jc = (k & 3) * _D
            u = (zu_v[jr, pl.ds(jc, _L)] * zi_v[jr, pl.ds(jc, _L)]
                 + zu_v[jr, pl.ds(jc + _L, _L)] * zi_v[jr, pl.ds(jc + _L, _L)]
                 + uy_v[jr, pl.ds(jc, _L)] * iy_v[jr, pl.ds(jc, _L)]
                 + uy_v[jr, pl.ds(jc + _L, _L)] * iy_v[jr, pl.ds(jc + _L, _L)])
            acc = jnp.where(lane == k, jnp.sum(u), acc)
        out_v[pl.ds(ch * _L, _L)] = acc
        return 0

    lax.fori_loop(0, _NCH, dot, 0)

    pltpu.sync_copy(out_v, out_hbm.at[pl.ds(base, _BPW)])


@jax.jit
def _mf(z, tu, ti, uy, iy):
    mesh = plsc.VectorSubcoreMesh(core_axis_name="c", subcore_axis_name="s")
    return pl.kernel(
        _body,
        mesh=mesh,
        compiler_params=pltpu.CompilerParams(
            needs_layout_passes=False, use_tc_tiling_on_sc=True,
            skip_device_barrier=True),
        out_type=jax.ShapeDtypeStruct((_B,), jnp.float32),
        scratch_types=[
            pltpu.VMEM((_BPW,), jnp.int32),
            pltpu.VMEM((_BPW,), jnp.int32),
            pltpu.VMEM((_BPW // 4, 128), jnp.float32),
            pltpu.VMEM((_BPW // 4, 128), jnp.float32),
            pltpu.VMEM((_BPW // 4, 128), jnp.float32),
            pltpu.VMEM((_BPW // 4, 128), jnp.float32),
            pltpu.VMEM((_BPW,), jnp.float32),
            pltpu.SemaphoreType.DMA,
        ],
    )(z, tu, ti, uy, iy)


def kernel(Z, Tu, Ti, uY, iY):
    return _mf(Z, Tu.astype(jnp.int32), Ti.astype(jnp.int32), uY, iY)


# speedup vs baseline: 1.1268x; 1.0078x over previous
import jax
import jax.numpy as jnp
from jax import lax
from jax.experimental import pallas as pl
from jax.experimental.pallas import tpu as pltpu
from jax.experimental.pallas import tpu_sc as plsc

_NU = 1000000
_NI = 100000
_D = 32
_B = 16384

_INFO = plsc.get_sparse_core_info()
_NC = _INFO.num_cores
_NS = _INFO.num_subcores
_NW = _NC * _NS
_BPW = _B // _NW
_L = 16
_NCH = _BPW // _L


def _body(z_hbm, tu_hbm, ti_hbm, uy_hbm, iy_hbm, d_hbm, out_hbm,
          tu_v, ti_v, zu_v, uy_v, zi_v, iy_v, out_v, sem):
    wid = lax.axis_index("s") * _NC + lax.axis_index("c")
    base = wid * _BPW

    pltpu.sync_copy(tu_hbm.at[pl.ds(base, _BPW)], tu_v)
    pltpu.sync_copy(ti_hbm.at[pl.ds(base, _BPW)], ti_v)

    def fire(ch, _):
        tu16 = tu_v[pl.ds(ch * _L, _L)]
        ti16 = ti_v[pl.ds(ch * _L, _L)]
        tz16 = ti16 + _NU
        for k in range(_L):
            r = tu16[k]
            i = ti16[k]
            iz = tz16[k]
            jr = ch * 4 + k // 4
            jc = (k & 3) * _D
            dst = (jr, pl.ds(jc, _D))
            pltpu.async_copy(z_hbm.at[r], zu_v.at[dst], sem)
            pltpu.async_copy(uy_hbm.at[r], uy_v.at[dst], sem)
            pltpu.async_copy(z_hbm.at[iz], zi_v.at[dst], sem)
            pltpu.async_copy(iy_hbm.at[i], iy_v.at[dst], sem)
        return 0

    lax.fori_loop(0, _NCH, fire, 0)

    pltpu.make_async_copy(d_hbm, zu_v, sem).wait()
    pltpu.make_async_copy(d_hbm, uy_v, sem).wait()
    pltpu.make_async_copy(d_hbm, zi_v, sem).wait()
    pltpu.make_async_copy(d_hbm, iy_v, sem).wait()

    lane = lax.broadcasted_iota(jnp.int32, (_L,), 0)

    def dot(ch, _):
        acc = jnp.zeros((_L,), jnp.float32)
        for k in range(_L):
            jr = ch * 4 + k // 4
            jc = (k & 3) * _D
            u = (zu_v[jr, pl.ds(jc, _L)] * zi_v[jr, pl.ds(jc, _L)]
                 + zu_v[jr, pl.ds(jc + _L, _L)] * zi_v[jr, pl.ds(jc + _L, _L)]
                 + uy_v[jr, pl.ds(jc, _L)] * iy_v[jr, pl.ds(jc, _L)]
                 + uy_v[jr, pl.ds(jc + _L, _L)] * iy_v[jr, pl.ds(jc + _L, _L)])
            acc = jnp.where(lane == k, jnp.sum(u), acc)
        out_v[pl.ds(ch * _L, _L)] = acc
        return 0

    lax.fori_loop(0, _NCH, dot, 0)

    pltpu.sync_copy(out_v, out_hbm.at[pl.ds(base, _BPW)])


@jax.jit
def _mf(z, tu, ti, uy, iy):
    dummy = jnp.zeros((_BPW // 4, 128), jnp.float32)
    mesh = plsc.VectorSubcoreMesh(core_axis_name="c", subcore_axis_name="s")
    return pl.kernel(
        _body,
        mesh=mesh,
        compiler_params=pltpu.CompilerParams(
            needs_layout_passes=False, use_tc_tiling_on_sc=True,
            skip_device_barrier=True),
        out_type=jax.ShapeDtypeStruct((_B,), jnp.float32),
        scratch_types=[
            pltpu.VMEM((_BPW,), jnp.int32),
            pltpu.VMEM((_BPW,), jnp.int32),
            pltpu.VMEM((_BPW // 4, 128), jnp.float32),
            pltpu.VMEM((_BPW // 4, 128), jnp.float32),
            pltpu.VMEM((_BPW // 4, 128), jnp.float32),
            pltpu.VMEM((_BPW // 4, 128), jnp.float32),
            pltpu.VMEM((_BPW,), jnp.float32),
            pltpu.SemaphoreType.DMA,
        ],
    )(z, tu, ti, uy, iy, dummy)


def kernel(Z, Tu, Ti, uY, iY):
    return _mf(Z, Tu.astype(jnp.int32), Ti.astype(jnp.int32), uY, iY)
